# baseline (speedup 1.0000x reference)
import functools

import jax
import jax.numpy as jnp
from jax import lax
from jax.experimental import pallas as pl
from jax.experimental.pallas import tpu as pltpu
from jax.experimental.pallas import tpu_sc as plsc

B, L = 4096, 200
VOCAB, EMB = 128 * 128, 32
N = B * L

NC, NS = 2, 16
NW = NC * NS
UNIT = 1024
NQ = B // UNIT
NUNITS = L * NQ
UNITS_PER_W = NUNITS // NW
IDX_ROWS = UNIT // 128
OUT_ROWS_TOTAL = N * EMB // 128
SLAB = 1024
EKS = EMB // 8
UNIT_OUT_ROWS = 256


def _tc_fold_table(emb_table, W, b1):
    def body(emb_ref, w_ref, b_ref, out_ref):
        w2 = w_ref[2:2 + EMB, :] * 2.0
        acc = jnp.dot(emb_ref[...], w2, preferred_element_type=jnp.float32)
        out_ref[...] = acc + 2.0 * b_ref[...]

    return pl.pallas_call(
        body,
        out_shape=jax.ShapeDtypeStruct((VOCAB, EMB), jnp.float32),
    )(emb_table, W, b1)


def _sc_gather_affine_tanh(t2, ids4d, lng3d, lat3d, wc2):
    mesh = plsc.VectorSubcoreMesh(core_axis_name="c", subcore_axis_name="s")

    @functools.partial(
        pl.kernel,
        out_type=jax.ShapeDtypeStruct((OUT_ROWS_TOTAL, 128), jnp.float32),
        mesh=mesh,
        scratch_types=[
            pltpu.VMEM((IDX_ROWS, 128), jnp.int32),
            pltpu.VMEM((UNIT, EMB), jnp.float32),
            pltpu.VMEM((UNIT_OUT_ROWS, 128), jnp.float32),
            pltpu.VMEM((UNIT,), jnp.float32),
            pltpu.VMEM((UNIT,), jnp.float32),
            pltpu.VMEM((2, EMB), jnp.float32),
            pltpu.SemaphoreType.DMA,
        ],
        compiler_params=pltpu.CompilerParams(
            use_tc_tiling_on_sc=False, needs_layout_passes=False),
    )
    def k(t2_hbm, ids_hbm, lng_hbm, lat_hbm, wc_hbm, out_hbm,
          idx_v, rows_v, out_v, lng_v, lat_v, wc_v, sem):
        wid = lax.axis_index("s") * NC + lax.axis_index("c")
        pltpu.sync_copy(wc_hbm, wc_v)
        w00 = wc_v[0, 0:16]
        w01 = wc_v[0, 16:32]
        w10 = wc_v[1, 0:16]
        w11 = wc_v[1, 16:32]
        ii = lax.iota(jnp.int32, 16)
        c_r0 = (ii // 8) * 64 + ii % 8
        c_r1 = c_r0 + 128

        def unit_body(uu, unit_carry):
            u = wid * UNITS_PER_W + uu
            l = u // NQ
            q = u % NQ
            pltpu.sync_copy(ids_hbm.at[l, q], idx_v)
            cps = [
                pltpu.async_copy(
                    t2_hbm.at[idx_v.at[j]],
                    rows_v.at[pl.ds(j * 128, 128)],
                    sem,
                )
                for j in range(IDX_ROWS)
            ]
            pltpu.sync_copy(lng_hbm.at[l, q], lng_v)
            pltpu.sync_copy(lat_hbm.at[l, q], lat_v)
            for cp in cps:
                cp.wait()

            def body(g, carry):
                tb = g * 16
                lng16 = lng_v[pl.ds(tb, 16)]
                lat16 = lat_v[pl.ds(tb, 16)]
                r0 = c_r0 + (g // 8) * 8
                r1 = c_r1 + (g // 8) * 8
                bi0 = (g % 8) * 16
                for j in range(16):
                    lng_s = lng16[j]
                    lat_s = lat16[j]
                    t = tb + j
                    col = jnp.full((16,), bi0 + j, jnp.int32)
                    g0 = rows_v[t, 0:16]
                    y0 = jnp.minimum(g0 + lng_s * w00 + lat_s * w10, 80.0)
                    e0 = jnp.exp(y0)
                    plsc.store_scatter(out_v, [r0, col],
                                       (e0 - 1.0) / (e0 + 1.0))
                    g1 = rows_v[t, 16:32]
                    y1 = jnp.minimum(g1 + lng_s * w01 + lat_s * w11, 80.0)
                    e1 = jnp.exp(y1)
                    plsc.store_scatter(out_v, [r1, col],
                                       (e1 - 1.0) / (e1 + 1.0))
                return carry

            lax.fori_loop(0, UNIT // 16, body, 0)
            for ek in range(EKS):
                dst_row = pl.multiple_of(l * SLAB + ek * 256 + q * 64, 64)
                pltpu.sync_copy(out_v.at[pl.ds(ek * 64, 64)],
                                out_hbm.at[pl.ds(dst_row, 64), :])
            return unit_carry

        lax.fori_loop(0, UNITS_PER_W, unit_body, 0)

    return k(t2, ids4d, lng3d, lat3d, wc2)


def kernel(lngs, lats, grid_id, emb_table, W, b):
    t2 = _tc_fold_table(emb_table, W, b.reshape(1, EMB))
    wc2 = W[0:2, :] * 2.0
    ids4d = grid_id.astype(jnp.int32).T.reshape(L, NQ, IDX_ROWS, 128)
    lng3d = lngs.T.reshape(L, NQ, UNIT)
    lat3d = lats.T.reshape(L, NQ, UNIT)
    out = _sc_gather_affine_tanh(t2, ids4d, lng3d, lat3d, wc2)
    o5 = out.reshape(L, EMB // 8, B // 128, 8, 128)
    return o5.transpose(2, 4, 0, 1, 3).reshape(B, L, EMB)
